# Initial kernel scaffold; baseline (speedup 1.0000x reference)
import functools

import jax
import jax.numpy as jnp
from jax import lax
from jax.experimental import pallas as pl
from jax.experimental.pallas import tpu as pltpu
from jax.experimental.pallas import tpu_sc as plsc

N_NODES = 10000
N_EDGES = 320000
IN_FEATS = 128
H_FEATS = 256
NUM_CLASSES = 64

NC = 2
NS = 16
NW = NC * NS
CHUNK = 128
EPW = 10240
NCHUNK = EPW // CHUNK
E_PAD = NW * EPW
ACC_ROWS = 10240
ZROWS_PER_TILE = ACC_ROWS // NS
OUT_ROWS_PER_TILE = N_NODES // NS


def _sc_agg_body(F, x_hbm, src_hbm, dst_hbm, ms0, ms1, dg0, dg1,
                 acc, dacc, src_v, dst_v, rows, ones_v, zbuf, zbuf_d, sem):
    c = lax.axis_index("c")
    s = lax.axis_index("s")
    wid = c * NS + s

    zero16 = jnp.zeros((16,), jnp.float32)
    one16 = jnp.ones((16,), jnp.float32)
    for i in range(16):
        for j in range(F // 16):
            zbuf[i, pl.ds(16 * j, 16)] = zero16
        zbuf_d[i, pl.ds(0, 16)] = zero16
    for i in range(CHUNK):
        ones_v[i, pl.ds(0, 16)] = one16

    def zloop(k, carry):
        base = s * ZROWS_PER_TILE + k * 16
        pltpu.sync_copy(zbuf, acc.at[pl.ds(base, 16)])
        pltpu.sync_copy(zbuf_d, dacc.at[pl.ds(base, 16)])
        return carry
    lax.fori_loop(0, ZROWS_PER_TILE // 16, zloop, 0)

    plsc.subcore_barrier()

    pltpu.sync_copy(src_hbm.at[wid], src_v)
    pltpu.sync_copy(dst_hbm.at[wid], dst_v)

    def eloop(j, carry):
        pltpu.async_copy(x_hbm.at[src_v.at[j]], rows, sem).wait()
        pltpu.sync_copy(rows, acc.at[dst_v.at[j]], add=True)
        pltpu.sync_copy(ones_v, dacc.at[dst_v.at[j]], add=True)
        return carry
    lax.fori_loop(0, NCHUNK, eloop, 0)

    plsc.subcore_barrier()

    r0 = s * OUT_ROWS_PER_TILE

    @pl.when(c == 0)
    def _():
        pltpu.sync_copy(acc.at[pl.ds(r0, OUT_ROWS_PER_TILE)],
                        ms0.at[pl.ds(r0, OUT_ROWS_PER_TILE)])
        pltpu.sync_copy(dacc.at[pl.ds(r0, OUT_ROWS_PER_TILE)],
                        dg0.at[pl.ds(r0, OUT_ROWS_PER_TILE)])

    @pl.when(c == 1)
    def _():
        pltpu.sync_copy(acc.at[pl.ds(r0, OUT_ROWS_PER_TILE)],
                        ms1.at[pl.ds(r0, OUT_ROWS_PER_TILE)])
        pltpu.sync_copy(dacc.at[pl.ds(r0, OUT_ROWS_PER_TILE)],
                        dg1.at[pl.ds(r0, OUT_ROWS_PER_TILE)])


def _make_sc_agg(F):
    mesh = plsc.VectorSubcoreMesh(core_axis_name="c", subcore_axis_name="s",
                                  num_cores=NC, num_subcores=NS)
    return pl.kernel(
        functools.partial(_sc_agg_body, F),
        out_type=[
            jax.ShapeDtypeStruct((N_NODES, F), jnp.float32),
            jax.ShapeDtypeStruct((N_NODES, F), jnp.float32),
            jax.ShapeDtypeStruct((N_NODES, 16), jnp.float32),
            jax.ShapeDtypeStruct((N_NODES, 16), jnp.float32),
        ],
        mesh=mesh,
        scratch_types=[
            pltpu.VMEM_SHARED((ACC_ROWS, F), jnp.float32),
            pltpu.VMEM_SHARED((ACC_ROWS, 16), jnp.float32),
            pltpu.VMEM((NCHUNK, CHUNK), jnp.int32),
            pltpu.VMEM((NCHUNK, CHUNK), jnp.int32),
            pltpu.VMEM((CHUNK, F), jnp.float32),
            pltpu.VMEM((CHUNK, 16), jnp.float32),
            pltpu.VMEM((16, F), jnp.float32),
            pltpu.VMEM((16, 16), jnp.float32),
            pltpu.SemaphoreType.DMA,
        ],
    )


_sc_agg_128 = _make_sc_agg(IN_FEATS)
_sc_agg_64 = _make_sc_agg(NUM_CLASSES)


def _tc1_body(x_ref, ms0_ref, ms1_ref, dg0_ref, dg1_ref,
              w1s_ref, w1n_ref, b1_ref, w2s_ref, w2n_ref, b2_ref,
              p2_ref, s2_ref):
    deg = jnp.maximum(dg0_ref[:, 0:1] + dg1_ref[:, 0:1], 1.0)
    h_n = (ms0_ref[...] + ms1_ref[...]) / deg
    h = (jnp.dot(x_ref[...], w1s_ref[...], preferred_element_type=jnp.float32)
         + jnp.dot(h_n, w1n_ref[...], preferred_element_type=jnp.float32)
         + b1_ref[...])
    h = jnp.maximum(h, 0.0)
    p2_ref[...] = jnp.dot(h, w2n_ref[...], preferred_element_type=jnp.float32)
    s2_ref[...] = (jnp.dot(h, w2s_ref[...], preferred_element_type=jnp.float32)
                   + b2_ref[...])


def _tc2_body(s2_ref, ms0_ref, ms1_ref, dg0_ref, dg1_ref, out_ref):
    deg = jnp.maximum(dg0_ref[:, 0:1] + dg1_ref[:, 0:1], 1.0)
    out_ref[...] = s2_ref[...] + (ms0_ref[...] + ms1_ref[...]) / deg


_TC_ROWS = 1000


def _tc1(x, ms0, ms1, dg0, dg1, w1s, w1n, b1, w2s, w2n, b2):
    grid = (N_NODES // _TC_ROWS,)
    row_block = lambda f: pl.BlockSpec((_TC_ROWS, f), lambda i: (i, 0))
    full = lambda a, b: pl.BlockSpec((a, b), lambda i: (0, 0))
    return pl.pallas_call(
        _tc1_body,
        grid=grid,
        in_specs=[
            row_block(IN_FEATS), row_block(IN_FEATS), row_block(IN_FEATS),
            row_block(16), row_block(16),
            full(IN_FEATS, H_FEATS), full(IN_FEATS, H_FEATS), full(1, H_FEATS),
            full(H_FEATS, NUM_CLASSES), full(H_FEATS, NUM_CLASSES),
            full(1, NUM_CLASSES),
        ],
        out_specs=[row_block(NUM_CLASSES), row_block(NUM_CLASSES)],
        out_shape=[
            jax.ShapeDtypeStruct((N_NODES, NUM_CLASSES), jnp.float32),
            jax.ShapeDtypeStruct((N_NODES, NUM_CLASSES), jnp.float32),
        ],
    )(x, ms0, ms1, dg0, dg1, w1s, w1n, b1, w2s, w2n, b2)


def _tc2(s2, ms0, ms1, dg0, dg1):
    grid = (N_NODES // _TC_ROWS,)
    row_block = lambda f: pl.BlockSpec((_TC_ROWS, f), lambda i: (i, 0))
    return pl.pallas_call(
        _tc2_body,
        grid=grid,
        in_specs=[
            row_block(NUM_CLASSES), row_block(NUM_CLASSES),
            row_block(NUM_CLASSES), row_block(16), row_block(16),
        ],
        out_specs=row_block(NUM_CLASSES),
        out_shape=jax.ShapeDtypeStruct((N_NODES, NUM_CLASSES), jnp.float32),
    )(s2, ms0, ms1, dg0, dg1)


def _pack_edges(edge_index):
    src = edge_index[0].astype(jnp.int32)
    dst = edge_index[1].astype(jnp.int32)
    pad = E_PAD - N_EDGES
    src = jnp.concatenate([src, jnp.zeros((pad,), jnp.int32)])
    dst = jnp.concatenate([dst, jnp.full((pad,), N_NODES, jnp.int32)])
    return src.reshape(NW, NCHUNK, CHUNK), dst.reshape(NW, NCHUNK, CHUNK)


def kernel(x, edge_index1, edge_index2, W1, b1, W2, b2):
    sp1, dp1 = _pack_edges(edge_index1)
    sp2, dp2 = _pack_edges(edge_index2)

    w1s = W1[:, :IN_FEATS].T
    w1n = W1[:, IN_FEATS:].T
    w2s = W2[:, :H_FEATS].T
    w2n = W2[:, H_FEATS:].T
    b1r = b1.reshape(1, H_FEATS)
    b2r = b2.reshape(1, NUM_CLASSES)

    ms10, ms11, dg10, dg11 = _sc_agg_128(x, sp1, dp1)
    p2, s2 = _tc1(x, ms10, ms11, dg10, dg11, w1s, w1n, b1r, w2s, w2n, b2r)
    ms20, ms21, dg20, dg21 = _sc_agg_64(p2, sp2, dp2)
    return _tc2(s2, ms20, ms21, dg20, dg21)


# speedup vs baseline: 4.6412x; 4.6412x over previous
import functools

import jax
import jax.numpy as jnp
from jax import lax
from jax.experimental import pallas as pl
from jax.experimental.pallas import tpu as pltpu
from jax.experimental.pallas import tpu_sc as plsc

N_NODES = 10000
N_EDGES = 320000
IN_FEATS = 128
H_FEATS = 256
NUM_CLASSES = 64

NC = 2
NS = 16
NW = NC * NS
CHUNK = 80
EPW = 10240
NCHUNK = EPW // CHUNK
E_PAD = NW * EPW
ACC_ROWS = 10112
ZROWS_PER_TILE = ACC_ROWS // NS
ZB = 8
IDX_STAGE = NCHUNK // 2


def _sc_agg_body(F, x_hbm, src_hbm, dst_hbm, ms0, ms1, dg0, dg1,
                 acc, dacc, src_v, dst_v, rows, ones_v, zbuf, zbuf_d, sem):
    c = lax.axis_index("c")
    s = lax.axis_index("s")
    wid = c * NS + s

    zero16 = jnp.zeros((16,), jnp.float32)
    one16 = jnp.ones((16,), jnp.float32)
    for i in range(ZB):
        for j in range(F // 16):
            zbuf[i, pl.ds(16 * j, 16)] = zero16
        zbuf_d[i, pl.ds(0, 16)] = zero16
    for i in range(CHUNK):
        ones_v[i, pl.ds(0, 16)] = one16

    def zloop(k, carry):
        base = s * ZROWS_PER_TILE + k * ZB
        pltpu.sync_copy(zbuf, acc.at[pl.ds(base, ZB)])
        pltpu.sync_copy(zbuf_d, dacc.at[pl.ds(base, ZB)])
        return carry
    lax.fori_loop(0, ZROWS_PER_TILE // ZB, zloop, 0)

    plsc.subcore_barrier()

    def eloop(j, carry):
        pltpu.async_copy(x_hbm.at[src_v.at[j]], rows, sem).wait()
        pltpu.sync_copy(rows, acc.at[dst_v.at[j]], add=True)
        pltpu.sync_copy(ones_v, dacc.at[dst_v.at[j]], add=True)
        return carry

    for h in range(NCHUNK // IDX_STAGE):
        pltpu.sync_copy(src_hbm.at[wid, pl.ds(h * IDX_STAGE, IDX_STAGE)], src_v)
        pltpu.sync_copy(dst_hbm.at[wid, pl.ds(h * IDX_STAGE, IDX_STAGE)], dst_v)
        lax.fori_loop(0, IDX_STAGE, eloop, 0)

    plsc.subcore_barrier()

    r0 = s * ZROWS_PER_TILE

    @pl.when(c == 0)
    def _():
        pltpu.sync_copy(acc.at[pl.ds(r0, ZROWS_PER_TILE)],
                        ms0.at[pl.ds(r0, ZROWS_PER_TILE)])
        pltpu.sync_copy(dacc.at[pl.ds(r0, ZROWS_PER_TILE)],
                        dg0.at[pl.ds(r0, ZROWS_PER_TILE)])

    @pl.when(c == 1)
    def _():
        pltpu.sync_copy(acc.at[pl.ds(r0, ZROWS_PER_TILE)],
                        ms1.at[pl.ds(r0, ZROWS_PER_TILE)])
        pltpu.sync_copy(dacc.at[pl.ds(r0, ZROWS_PER_TILE)],
                        dg1.at[pl.ds(r0, ZROWS_PER_TILE)])


def _make_sc_agg(F):
    mesh = plsc.VectorSubcoreMesh(core_axis_name="c", subcore_axis_name="s",
                                  num_cores=NC, num_subcores=NS)
    return pl.kernel(
        functools.partial(_sc_agg_body, F),
        out_type=[
            jax.ShapeDtypeStruct((ACC_ROWS, F), jnp.float32),
            jax.ShapeDtypeStruct((ACC_ROWS, F), jnp.float32),
            jax.ShapeDtypeStruct((ACC_ROWS, 16), jnp.float32),
            jax.ShapeDtypeStruct((ACC_ROWS, 16), jnp.float32),
        ],
        mesh=mesh,
        scratch_types=[
            pltpu.VMEM_SHARED((ACC_ROWS, F), jnp.float32),
            pltpu.VMEM_SHARED((ACC_ROWS, 16), jnp.float32),
            pltpu.VMEM((IDX_STAGE, CHUNK), jnp.int32),
            pltpu.VMEM((IDX_STAGE, CHUNK), jnp.int32),
            pltpu.VMEM((CHUNK, F), jnp.float32),
            pltpu.VMEM((CHUNK, 16), jnp.float32),
            pltpu.VMEM((ZB, F), jnp.float32),
            pltpu.VMEM((ZB, 16), jnp.float32),
            pltpu.SemaphoreType.DMA,
        ],
        compiler_params=pltpu.CompilerParams(use_tc_tiling_on_sc=False),
    )


_sc_agg_128 = _make_sc_agg(IN_FEATS)
_sc_agg_64 = _make_sc_agg(NUM_CLASSES)


def _tc1_body(x_ref, ms0_ref, ms1_ref, dg0_ref, dg1_ref,
              w1s_ref, w1n_ref, b1_ref, w2s_ref, w2n_ref, b2_ref,
              p2_ref, s2_ref):
    deg = jnp.maximum(dg0_ref[:, 0:1] + dg1_ref[:, 0:1], 1.0)
    h_n = (ms0_ref[...] + ms1_ref[...]) / deg
    h = (jnp.dot(x_ref[...], w1s_ref[...], preferred_element_type=jnp.float32)
         + jnp.dot(h_n, w1n_ref[...], preferred_element_type=jnp.float32)
         + b1_ref[...])
    h = jnp.maximum(h, 0.0)
    p2_ref[...] = jnp.dot(h, w2n_ref[...], preferred_element_type=jnp.float32)
    s2_ref[...] = (jnp.dot(h, w2s_ref[...], preferred_element_type=jnp.float32)
                   + b2_ref[...])


def _tc2_body(s2_ref, ms0_ref, ms1_ref, dg0_ref, dg1_ref, out_ref):
    deg = jnp.maximum(dg0_ref[:, 0:1] + dg1_ref[:, 0:1], 1.0)
    out_ref[...] = s2_ref[...] + (ms0_ref[...] + ms1_ref[...]) / deg


_TC_ROWS = 1000


def _tc1(x, ms0, ms1, dg0, dg1, w1s, w1n, b1, w2s, w2n, b2):
    grid = (N_NODES // _TC_ROWS,)
    row_block = lambda f: pl.BlockSpec((_TC_ROWS, f), lambda i: (i, 0))
    full = lambda a, b: pl.BlockSpec((a, b), lambda i: (0, 0))
    return pl.pallas_call(
        _tc1_body,
        grid=grid,
        in_specs=[
            row_block(IN_FEATS), row_block(IN_FEATS), row_block(IN_FEATS),
            row_block(16), row_block(16),
            full(IN_FEATS, H_FEATS), full(IN_FEATS, H_FEATS), full(1, H_FEATS),
            full(H_FEATS, NUM_CLASSES), full(H_FEATS, NUM_CLASSES),
            full(1, NUM_CLASSES),
        ],
        out_specs=[row_block(NUM_CLASSES), row_block(NUM_CLASSES)],
        out_shape=[
            jax.ShapeDtypeStruct((N_NODES, NUM_CLASSES), jnp.float32),
            jax.ShapeDtypeStruct((N_NODES, NUM_CLASSES), jnp.float32),
        ],
    )(x, ms0, ms1, dg0, dg1, w1s, w1n, b1, w2s, w2n, b2)


def _tc2(s2, ms0, ms1, dg0, dg1):
    grid = (N_NODES // _TC_ROWS,)
    row_block = lambda f: pl.BlockSpec((_TC_ROWS, f), lambda i: (i, 0))
    return pl.pallas_call(
        _tc2_body,
        grid=grid,
        in_specs=[
            row_block(NUM_CLASSES), row_block(NUM_CLASSES),
            row_block(NUM_CLASSES), row_block(16), row_block(16),
        ],
        out_specs=row_block(NUM_CLASSES),
        out_shape=jax.ShapeDtypeStruct((N_NODES, NUM_CLASSES), jnp.float32),
    )(s2, ms0, ms1, dg0, dg1)


def _pack_edges(edge_index):
    src = edge_index[0].astype(jnp.int32)
    dst = edge_index[1].astype(jnp.int32)
    pad = E_PAD - N_EDGES
    src = jnp.concatenate([src, jnp.zeros((pad,), jnp.int32)])
    dst = jnp.concatenate([dst, jnp.full((pad,), N_NODES, jnp.int32)])
    return src.reshape(NW, NCHUNK, CHUNK), dst.reshape(NW, NCHUNK, CHUNK)


def kernel(x, edge_index1, edge_index2, W1, b1, W2, b2):
    sp1, dp1 = _pack_edges(edge_index1)
    sp2, dp2 = _pack_edges(edge_index2)

    w1s = W1[:, :IN_FEATS].T
    w1n = W1[:, IN_FEATS:].T
    w2s = W2[:, :H_FEATS].T
    w2n = W2[:, H_FEATS:].T
    b1r = b1.reshape(1, H_FEATS)
    b2r = b2.reshape(1, NUM_CLASSES)

    ms10, ms11, dg10, dg11 = _sc_agg_128(x, sp1, dp1)
    p2, s2 = _tc1(x, ms10, ms11, dg10, dg11, w1s, w1n, b1r, w2s, w2n, b2r)
    ms20, ms21, dg20, dg21 = _sc_agg_64(p2, sp2, dp2)
    return _tc2(s2, ms20, ms21, dg20, dg21)


# speedup vs baseline: 5.3411x; 1.1508x over previous
import functools

import jax
import jax.numpy as jnp
from jax import lax
from jax.experimental import pallas as pl
from jax.experimental.pallas import tpu as pltpu
from jax.experimental.pallas import tpu_sc as plsc

N_NODES = 10000
N_EDGES = 320000
IN_FEATS = 128
H_FEATS = 256
NUM_CLASSES = 64

NC = 2
NS = 16
NW = NC * NS
CHUNK = 80
EPW = 10240
NCHUNK = EPW // CHUNK
E_PAD = NW * EPW
ACC_ROWS = 10112
ZROWS_PER_TILE = ACC_ROWS // NS
IDX_STAGE = 32
N_STAGES = NCHUNK // IDX_STAGE


def _sc_agg_body(F, x_hbm, src_hbm, dst_hbm, z_big, z_deg, ms0, ms1, dg0, dg1,
                 acc, dacc, src_v, dst_v, rows0, rows1, ones_v,
                 gsem0, gsem1, dsem):
    c = lax.axis_index("c")
    s = lax.axis_index("s")
    wid = c * NS + s

    one16 = jnp.ones((16,), jnp.float32)
    for i in range(CHUNK):
        ones_v[i, pl.ds(0, 16)] = one16

    r0 = s * ZROWS_PER_TILE
    pltpu.sync_copy(z_big, acc.at[pl.ds(r0, ZROWS_PER_TILE)])
    pltpu.sync_copy(z_deg, dacc.at[pl.ds(r0, ZROWS_PER_TILE)])

    plsc.subcore_barrier()

    def stage_loop(k, carry):
        j0 = 2 * k
        j1 = j0 + 1
        pltpu.make_async_copy(x_hbm.at[src_v.at[j0]], rows0, gsem0).wait()
        d0 = pltpu.async_copy(ones_v, dacc.at[dst_v.at[j0]], dsem, add=True)
        pltpu.sync_copy(rows0, acc.at[dst_v.at[j0]], add=True)

        @pl.when(k < IDX_STAGE // 2 - 1)
        def _():
            pltpu.async_copy(x_hbm.at[src_v.at[j0 + 2]], rows0, gsem0)

        pltpu.make_async_copy(x_hbm.at[src_v.at[j1]], rows1, gsem1).wait()
        d1 = pltpu.async_copy(ones_v, dacc.at[dst_v.at[j1]], dsem, add=True)
        pltpu.sync_copy(rows1, acc.at[dst_v.at[j1]], add=True)

        @pl.when(k < IDX_STAGE // 2 - 1)
        def _():
            pltpu.async_copy(x_hbm.at[src_v.at[j1 + 2]], rows1, gsem1)

        d0.wait()
        d1.wait()
        return carry

    for h in range(N_STAGES):
        pltpu.sync_copy(src_hbm.at[wid, pl.ds(h * IDX_STAGE, IDX_STAGE)], src_v)
        pltpu.sync_copy(dst_hbm.at[wid, pl.ds(h * IDX_STAGE, IDX_STAGE)], dst_v)
        pltpu.async_copy(x_hbm.at[src_v.at[0]], rows0, gsem0)
        pltpu.async_copy(x_hbm.at[src_v.at[1]], rows1, gsem1)
        lax.fori_loop(0, IDX_STAGE // 2, stage_loop, 0)

    plsc.subcore_barrier()

    r0 = s * ZROWS_PER_TILE

    @pl.when(c == 0)
    def _():
        pltpu.sync_copy(acc.at[pl.ds(r0, ZROWS_PER_TILE)],
                        ms0.at[pl.ds(r0, ZROWS_PER_TILE)])
        pltpu.sync_copy(dacc.at[pl.ds(r0, ZROWS_PER_TILE)],
                        dg0.at[pl.ds(r0, ZROWS_PER_TILE)])

    @pl.when(c == 1)
    def _():
        pltpu.sync_copy(acc.at[pl.ds(r0, ZROWS_PER_TILE)],
                        ms1.at[pl.ds(r0, ZROWS_PER_TILE)])
        pltpu.sync_copy(dacc.at[pl.ds(r0, ZROWS_PER_TILE)],
                        dg1.at[pl.ds(r0, ZROWS_PER_TILE)])


def _make_sc_agg(F):
    mesh = plsc.VectorSubcoreMesh(core_axis_name="c", subcore_axis_name="s",
                                  num_cores=NC, num_subcores=NS)
    return pl.kernel(
        functools.partial(_sc_agg_body, F),
        out_type=[
            jax.ShapeDtypeStruct((ACC_ROWS, F), jnp.float32),
            jax.ShapeDtypeStruct((ACC_ROWS, F), jnp.float32),
            jax.ShapeDtypeStruct((ACC_ROWS, 16), jnp.float32),
            jax.ShapeDtypeStruct((ACC_ROWS, 16), jnp.float32),
        ],
        mesh=mesh,
        scratch_types=[
            pltpu.VMEM_SHARED((ACC_ROWS, F), jnp.float32),
            pltpu.VMEM_SHARED((ACC_ROWS, 16), jnp.float32),
            pltpu.VMEM((IDX_STAGE, CHUNK), jnp.int32),
            pltpu.VMEM((IDX_STAGE, CHUNK), jnp.int32),
            pltpu.VMEM((CHUNK, F), jnp.float32),
            pltpu.VMEM((CHUNK, F), jnp.float32),
            pltpu.VMEM((CHUNK, 16), jnp.float32),
            pltpu.SemaphoreType.DMA,
            pltpu.SemaphoreType.DMA,
            pltpu.SemaphoreType.DMA,
        ],
        compiler_params=pltpu.CompilerParams(use_tc_tiling_on_sc=False),
    )


_sc_agg_128 = _make_sc_agg(IN_FEATS)
_sc_agg_64 = _make_sc_agg(NUM_CLASSES)


def _tc1_body(x_ref, ms0_ref, ms1_ref, dg0_ref, dg1_ref,
              w1s_ref, w1n_ref, b1_ref, w2s_ref, w2n_ref, b2_ref,
              p2_ref, s2_ref):
    deg = jnp.maximum(dg0_ref[:, 0:1] + dg1_ref[:, 0:1], 1.0)
    h_n = (ms0_ref[...] + ms1_ref[...]) / deg
    h = (jnp.dot(x_ref[...], w1s_ref[...], preferred_element_type=jnp.float32)
         + jnp.dot(h_n, w1n_ref[...], preferred_element_type=jnp.float32)
         + b1_ref[...])
    h = jnp.maximum(h, 0.0)
    p2_ref[...] = jnp.dot(h, w2n_ref[...], preferred_element_type=jnp.float32)
    s2_ref[...] = (jnp.dot(h, w2s_ref[...], preferred_element_type=jnp.float32)
                   + b2_ref[...])


def _tc2_body(s2_ref, ms0_ref, ms1_ref, dg0_ref, dg1_ref, out_ref):
    deg = jnp.maximum(dg0_ref[:, 0:1] + dg1_ref[:, 0:1], 1.0)
    out_ref[...] = s2_ref[...] + (ms0_ref[...] + ms1_ref[...]) / deg


_TC_ROWS = 1000


def _tc1(x, ms0, ms1, dg0, dg1, w1s, w1n, b1, w2s, w2n, b2):
    grid = (N_NODES // _TC_ROWS,)
    row_block = lambda f: pl.BlockSpec((_TC_ROWS, f), lambda i: (i, 0))
    full = lambda a, b: pl.BlockSpec((a, b), lambda i: (0, 0))
    return pl.pallas_call(
        _tc1_body,
        grid=grid,
        in_specs=[
            row_block(IN_FEATS), row_block(IN_FEATS), row_block(IN_FEATS),
            row_block(16), row_block(16),
            full(IN_FEATS, H_FEATS), full(IN_FEATS, H_FEATS), full(1, H_FEATS),
            full(H_FEATS, NUM_CLASSES), full(H_FEATS, NUM_CLASSES),
            full(1, NUM_CLASSES),
        ],
        out_specs=[row_block(NUM_CLASSES), row_block(NUM_CLASSES)],
        out_shape=[
            jax.ShapeDtypeStruct((N_NODES, NUM_CLASSES), jnp.float32),
            jax.ShapeDtypeStruct((N_NODES, NUM_CLASSES), jnp.float32),
        ],
    )(x, ms0, ms1, dg0, dg1, w1s, w1n, b1, w2s, w2n, b2)


def _tc2(s2, ms0, ms1, dg0, dg1):
    grid = (N_NODES // _TC_ROWS,)
    row_block = lambda f: pl.BlockSpec((_TC_ROWS, f), lambda i: (i, 0))
    return pl.pallas_call(
        _tc2_body,
        grid=grid,
        in_specs=[
            row_block(NUM_CLASSES), row_block(NUM_CLASSES),
            row_block(NUM_CLASSES), row_block(16), row_block(16),
        ],
        out_specs=row_block(NUM_CLASSES),
        out_shape=jax.ShapeDtypeStruct((N_NODES, NUM_CLASSES), jnp.float32),
    )(s2, ms0, ms1, dg0, dg1)


def _pack_edges(edge_index):
    src = edge_index[0].astype(jnp.int32)
    dst = edge_index[1].astype(jnp.int32)
    pad = E_PAD - N_EDGES
    src = jnp.concatenate([src, jnp.zeros((pad,), jnp.int32)])
    dst = jnp.concatenate([dst, jnp.full((pad,), N_NODES, jnp.int32)])
    return src.reshape(NW, NCHUNK, CHUNK), dst.reshape(NW, NCHUNK, CHUNK)


def kernel(x, edge_index1, edge_index2, W1, b1, W2, b2):
    sp1, dp1 = _pack_edges(edge_index1)
    sp2, dp2 = _pack_edges(edge_index2)

    w1s = W1[:, :IN_FEATS].T
    w1n = W1[:, IN_FEATS:].T
    w2s = W2[:, :H_FEATS].T
    w2n = W2[:, H_FEATS:].T
    b1r = b1.reshape(1, H_FEATS)
    b2r = b2.reshape(1, NUM_CLASSES)

    z128 = jnp.zeros((ZROWS_PER_TILE, IN_FEATS), jnp.float32)
    z64 = jnp.zeros((ZROWS_PER_TILE, NUM_CLASSES), jnp.float32)
    z16 = jnp.zeros((ZROWS_PER_TILE, 16), jnp.float32)

    ms10, ms11, dg10, dg11 = _sc_agg_128(x, sp1, dp1, z128, z16)
    p2, s2 = _tc1(x, ms10, ms11, dg10, dg11, w1s, w1n, b1r, w2s, w2n, b2r)
    ms20, ms21, dg20, dg21 = _sc_agg_64(p2, sp2, dp2, z64, z16)
    return _tc2(s2, ms20, ms21, dg20, dg21)


# speedup vs baseline: 6.1975x; 1.1603x over previous
import functools

import jax
import jax.numpy as jnp
from jax import lax
from jax.experimental import pallas as pl
from jax.experimental.pallas import tpu as pltpu
from jax.experimental.pallas import tpu_sc as plsc

N_NODES = 10000
N_EDGES = 320000
IN_FEATS = 128
H_FEATS = 256
NUM_CLASSES = 64

NC = 2
NS = 16
NW = NC * NS
CHUNK = 80
E_PAD = 327680
TOTAL_CHUNKS = E_PAD // CHUNK
ACC_ROWS = 10112
ZROWS_PER_TILE = ACC_ROWS // NS
IDX_STAGE = 32
CORE0_CHUNKS = 224
CORE1_CHUNKS = 32
assert NS * (CORE0_CHUNKS + CORE1_CHUNKS) == TOTAL_CHUNKS


def _sc_agg_body(F, x_hbm, src_hbm, dst_hbm, z_big, z_deg, ms0, ms1, dg0, dg1,
                 acc, dacc, src_v, dst_v, rows0, rows1, ones_v,
                 gsem0, gsem1, dsem):
    c = lax.axis_index("c")
    s = lax.axis_index("s")

    one16 = jnp.ones((16,), jnp.float32)
    for i in range(CHUNK):
        ones_v[i, pl.ds(0, 16)] = one16

    r0 = s * ZROWS_PER_TILE
    pltpu.sync_copy(z_big, rows0)
    for k in range(ZROWS_PER_TILE // CHUNK):
        pltpu.sync_copy(rows0, acc.at[pl.ds(r0 + k * CHUNK, CHUNK)])
    rem = ZROWS_PER_TILE % CHUNK
    pltpu.sync_copy(rows0.at[pl.ds(0, rem)],
                    acc.at[pl.ds(r0 + ZROWS_PER_TILE - rem, rem)])
    pltpu.sync_copy(z_deg, dacc.at[pl.ds(r0, ZROWS_PER_TILE)])

    plsc.subcore_barrier()

    tile_base = jnp.where(c == 0, s * CORE0_CHUNKS,
                          NS * CORE0_CHUNKS + s * CORE1_CHUNKS)
    n_stages = jnp.where(c == 0, CORE0_CHUNKS // IDX_STAGE,
                         CORE1_CHUNKS // IDX_STAGE)

    def chunk_loop(k, carry):
        j0 = 2 * k
        j1 = j0 + 1
        pltpu.make_async_copy(x_hbm.at[src_v.at[j0]], rows0, gsem0).wait()
        d0 = pltpu.async_copy(ones_v, dacc.at[dst_v.at[j0]], dsem, add=True)
        pltpu.sync_copy(rows0, acc.at[dst_v.at[j0]], add=True)

        @pl.when(k < IDX_STAGE // 2 - 1)
        def _():
            pltpu.async_copy(x_hbm.at[src_v.at[j0 + 2]], rows0, gsem0)

        pltpu.make_async_copy(x_hbm.at[src_v.at[j1]], rows1, gsem1).wait()
        d1 = pltpu.async_copy(ones_v, dacc.at[dst_v.at[j1]], dsem, add=True)
        pltpu.sync_copy(rows1, acc.at[dst_v.at[j1]], add=True)

        @pl.when(k < IDX_STAGE // 2 - 1)
        def _():
            pltpu.async_copy(x_hbm.at[src_v.at[j1 + 2]], rows1, gsem1)

        d0.wait()
        d1.wait()
        return carry

    def stage_loop(h, carry):
        row0 = tile_base + h * IDX_STAGE
        pltpu.sync_copy(src_hbm.at[pl.ds(row0, IDX_STAGE)], src_v)
        pltpu.sync_copy(dst_hbm.at[pl.ds(row0, IDX_STAGE)], dst_v)
        pltpu.async_copy(x_hbm.at[src_v.at[0]], rows0, gsem0)
        pltpu.async_copy(x_hbm.at[src_v.at[1]], rows1, gsem1)
        lax.fori_loop(0, IDX_STAGE // 2, chunk_loop, 0)
        return carry

    lax.fori_loop(0, n_stages, stage_loop, 0)

    plsc.subcore_barrier()

    r0 = s * ZROWS_PER_TILE

    @pl.when(c == 0)
    def _():
        pltpu.sync_copy(acc.at[pl.ds(r0, ZROWS_PER_TILE)],
                        ms0.at[pl.ds(r0, ZROWS_PER_TILE)])
        pltpu.sync_copy(dacc.at[pl.ds(r0, ZROWS_PER_TILE)],
                        dg0.at[pl.ds(r0, ZROWS_PER_TILE)])

    @pl.when(c == 1)
    def _():
        pltpu.sync_copy(acc.at[pl.ds(r0, ZROWS_PER_TILE)],
                        ms1.at[pl.ds(r0, ZROWS_PER_TILE)])
        pltpu.sync_copy(dacc.at[pl.ds(r0, ZROWS_PER_TILE)],
                        dg1.at[pl.ds(r0, ZROWS_PER_TILE)])


def _make_sc_agg(F):
    mesh = plsc.VectorSubcoreMesh(core_axis_name="c", subcore_axis_name="s",
                                  num_cores=NC, num_subcores=NS)
    return pl.kernel(
        functools.partial(_sc_agg_body, F),
        out_type=[
            jax.ShapeDtypeStruct((ACC_ROWS, F), jnp.float32),
            jax.ShapeDtypeStruct((ACC_ROWS, F), jnp.float32),
            jax.ShapeDtypeStruct((ACC_ROWS, 16), jnp.float32),
            jax.ShapeDtypeStruct((ACC_ROWS, 16), jnp.float32),
        ],
        mesh=mesh,
        scratch_types=[
            pltpu.VMEM_SHARED((ACC_ROWS, F), jnp.float32),
            pltpu.VMEM_SHARED((ACC_ROWS, 16), jnp.float32),
            pltpu.VMEM((IDX_STAGE, CHUNK), jnp.int32),
            pltpu.VMEM((IDX_STAGE, CHUNK), jnp.int32),
            pltpu.VMEM((CHUNK, F), jnp.float32),
            pltpu.VMEM((CHUNK, F), jnp.float32),
            pltpu.VMEM((CHUNK, 16), jnp.float32),
            pltpu.SemaphoreType.DMA,
            pltpu.SemaphoreType.DMA,
            pltpu.SemaphoreType.DMA,
        ],
        compiler_params=pltpu.CompilerParams(use_tc_tiling_on_sc=False),
    )


_sc_agg_128 = _make_sc_agg(IN_FEATS)
_sc_agg_64 = _make_sc_agg(NUM_CLASSES)


def _tc1_body(x_ref, ms0_ref, ms1_ref, dg0_ref, dg1_ref,
              w1s_ref, w1n_ref, b1_ref, w2s_ref, w2n_ref, b2_ref,
              p2_ref, s2_ref):
    deg = jnp.maximum(dg0_ref[:, 0:1] + dg1_ref[:, 0:1], 1.0)
    h_n = (ms0_ref[...] + ms1_ref[...]) / deg
    h = (jnp.dot(x_ref[...], w1s_ref[...], preferred_element_type=jnp.float32)
         + jnp.dot(h_n, w1n_ref[...], preferred_element_type=jnp.float32)
         + b1_ref[...])
    h = jnp.maximum(h, 0.0)
    p2_ref[...] = jnp.dot(h, w2n_ref[...], preferred_element_type=jnp.float32)
    s2_ref[...] = (jnp.dot(h, w2s_ref[...], preferred_element_type=jnp.float32)
                   + b2_ref[...])


def _tc2_body(s2_ref, ms0_ref, ms1_ref, dg0_ref, dg1_ref, out_ref):
    deg = jnp.maximum(dg0_ref[:, 0:1] + dg1_ref[:, 0:1], 1.0)
    out_ref[...] = s2_ref[...] + (ms0_ref[...] + ms1_ref[...]) / deg


_TC_ROWS = 1000


def _tc1(x, ms0, ms1, dg0, dg1, w1s, w1n, b1, w2s, w2n, b2):
    grid = (N_NODES // _TC_ROWS,)
    row_block = lambda f: pl.BlockSpec((_TC_ROWS, f), lambda i: (i, 0))
    full = lambda a, b: pl.BlockSpec((a, b), lambda i: (0, 0))
    return pl.pallas_call(
        _tc1_body,
        grid=grid,
        in_specs=[
            row_block(IN_FEATS), row_block(IN_FEATS), row_block(IN_FEATS),
            row_block(16), row_block(16),
            full(IN_FEATS, H_FEATS), full(IN_FEATS, H_FEATS), full(1, H_FEATS),
            full(H_FEATS, NUM_CLASSES), full(H_FEATS, NUM_CLASSES),
            full(1, NUM_CLASSES),
        ],
        out_specs=[row_block(NUM_CLASSES), row_block(NUM_CLASSES)],
        out_shape=[
            jax.ShapeDtypeStruct((N_NODES, NUM_CLASSES), jnp.float32),
            jax.ShapeDtypeStruct((N_NODES, NUM_CLASSES), jnp.float32),
        ],
    )(x, ms0, ms1, dg0, dg1, w1s, w1n, b1, w2s, w2n, b2)


def _tc2(s2, ms0, ms1, dg0, dg1):
    grid = (N_NODES // _TC_ROWS,)
    row_block = lambda f: pl.BlockSpec((_TC_ROWS, f), lambda i: (i, 0))
    return pl.pallas_call(
        _tc2_body,
        grid=grid,
        in_specs=[
            row_block(NUM_CLASSES), row_block(NUM_CLASSES),
            row_block(NUM_CLASSES), row_block(16), row_block(16),
        ],
        out_specs=row_block(NUM_CLASSES),
        out_shape=jax.ShapeDtypeStruct((N_NODES, NUM_CLASSES), jnp.float32),
    )(s2, ms0, ms1, dg0, dg1)


def _pack_edges(edge_index):
    src = edge_index[0].astype(jnp.int32)
    dst = edge_index[1].astype(jnp.int32)
    pad = E_PAD - N_EDGES
    src = jnp.concatenate([src, jnp.zeros((pad,), jnp.int32)])
    dst = jnp.concatenate([dst, jnp.full((pad,), N_NODES, jnp.int32)])
    return src.reshape(TOTAL_CHUNKS, CHUNK), dst.reshape(TOTAL_CHUNKS, CHUNK)


def kernel(x, edge_index1, edge_index2, W1, b1, W2, b2):
    sp1, dp1 = _pack_edges(edge_index1)
    sp2, dp2 = _pack_edges(edge_index2)

    w1s = W1[:, :IN_FEATS].T
    w1n = W1[:, IN_FEATS:].T
    w2s = W2[:, :H_FEATS].T
    w2n = W2[:, H_FEATS:].T
    b1r = b1.reshape(1, H_FEATS)
    b2r = b2.reshape(1, NUM_CLASSES)

    z128 = jnp.zeros((CHUNK, IN_FEATS), jnp.float32)
    z64 = jnp.zeros((CHUNK, NUM_CLASSES), jnp.float32)
    z16 = jnp.zeros((ZROWS_PER_TILE, 16), jnp.float32)

    ms10, ms11, dg10, dg11 = _sc_agg_128(x, sp1, dp1, z128, z16)
    p2, s2 = _tc1(x, ms10, ms11, dg10, dg11, w1s, w1n, b1r, w2s, w2n, b2r)
    ms20, ms21, dg20, dg21 = _sc_agg_64(p2, sp2, dp2, z64, z16)
    return _tc2(s2, ms20, ms21, dg20, dg21)
